# Initial kernel scaffold; baseline (speedup 1.0000x reference)
import jax
import jax.numpy as jnp
from jax.experimental import pallas as pl


def kernel(xyz, feats, params):
    raise NotImplementedError("write your pallas kernel here")


# speedup vs baseline: 1.4909x; 1.4909x over previous
import functools

import jax
import jax.numpy as jnp
import numpy as np
from jax import lax
from jax.experimental import pallas as pl
from jax.experimental.pallas import tpu as pltpu

_B, _N, _IN_CH, _OUT_CH = 2, 4096, 3, 512
_SA = [
    ('sa1', 1024, [0.02, 0.04], [32, 32]),
    ('sa2', 256, [0.04, 0.08], [32, 32]),
    ('sa3', 64, [0.08, 0.16], [32, 32]),
    ('sa4', 16, [0.16, 0.32], [32, 32]),
]


def _fps_body(npoint, n, S, rows_ref, xt_ref, out_ref):
    x2 = xt_ref[0, 0]
    y2 = xt_ref[0, 1]
    z2 = xt_ref[0, 2]
    iota = (lax.broadcasted_iota(jnp.int32, (S, 128), 0) * 128
            + lax.broadcasted_iota(jnp.int32, (S, 128), 1))
    out_ref[0, pl.ds(0, 1), :] = rows_ref[0, pl.ds(0, 1), :]

    def step(i, carry):
        dists, last = carry
        p = rows_ref[0, pl.ds(last, 1), :]
        px = p[0:1, 0:1]
        py = p[0:1, 1:2]
        pz = p[0:1, 2:3]
        dx = x2 - px
        dy = y2 - py
        dz = z2 - pz
        d = (dx * dx + dy * dy) + dz * dz
        dists = jnp.minimum(dists, d)
        m = jnp.max(dists)
        sel = jnp.where(dists == m, iota, jnp.int32(S * 128))
        nxt = jnp.min(sel)
        out_ref[0, pl.ds(i, 1), :] = rows_ref[0, pl.ds(nxt, 1), :]
        return dists, nxt

    dists0 = jnp.full((S, 128), 1e10, jnp.float32)
    lax.fori_loop(1, npoint, step, (dists0, jnp.int32(0)))


def _fps_level(rows, npoint):
    b, n, _ = rows.shape
    S = max(1, n // 128)
    npad = S * 128
    t = jnp.transpose(rows, (0, 2, 1))
    if npad > n:
        t = jnp.concatenate(
            [t, jnp.broadcast_to(t[:, :, 0:1], (b, 3, npad - n))], axis=2)
    xt = t.reshape(b, 3, S, 128)
    return pl.pallas_call(
        functools.partial(_fps_body, npoint, n, S),
        grid=(b,),
        in_specs=[
            pl.BlockSpec((1, n, 3), lambda i: (i, 0, 0)),
            pl.BlockSpec((1, 3, S, 128), lambda i: (i, 0, 0, 0)),
        ],
        out_specs=pl.BlockSpec((1, npoint, 3), lambda i: (i, 0, 0)),
        out_shape=jax.ShapeDtypeStruct((b, npoint, 3), jnp.float32),
    )(rows, xt)


def _bn(x, g, beta, axes):
    mean = jnp.mean(x, axis=axes, keepdims=True)
    var = jnp.var(x, axis=axes, keepdims=True)
    shape = [1] * x.ndim
    shape[1] = -1
    return (x - mean) / jnp.sqrt(var + 1e-5) * g.reshape(shape) + beta.reshape(shape)


def _conv_bn(x, p, relu=True):
    y = jnp.einsum('oc,bc...->bo...', p['W'], x) + p['b'].reshape((1, -1) + (1,) * (x.ndim - 2))
    y = _bn(y, p['g'], p['beta'], (0,) + tuple(range(2, x.ndim)))
    return jax.nn.relu(y) if relu else y


def _gather(pts, idx):
    return jax.vmap(lambda p, i: p[i])(pts, idx)


def _ball_query(radius, nsample, xyz, new_xyz):
    d2 = jnp.sum((new_xyz[:, :, None, :] - xyz[:, None, :, :]) ** 2, axis=-1)
    n = xyz.shape[1]
    k = jnp.where(d2 < radius * radius, jnp.arange(n, dtype=jnp.int32)[None, None, :], jnp.int32(n))
    idx = -lax.top_k(-k, nsample)[0]
    first = idx[..., :1]
    return jnp.where(idx >= n, first, idx)


def _sa_stage(xyz, feats, branches, npoint, radii, nsamples):
    new_xyz = _fps_level(xyz, npoint)
    feats_nl = jnp.transpose(feats, (0, 2, 1))
    outs = []
    for r, ns, branch in zip(radii, nsamples, branches):
        idx = _ball_query(r, ns, xyz, new_xyz)
        gxyz = _gather(xyz, idx) - new_xyz[:, :, None, :]
        gfeat = _gather(feats_nl, idx)
        g = jnp.transpose(jnp.concatenate([gxyz, gfeat], axis=-1), (0, 3, 1, 2))
        for lp in branch:
            g = _conv_bn(g, lp, relu=True)
        outs.append(jnp.max(g, axis=3))
    return new_xyz, jnp.concatenate(outs, axis=1)


def _fp_stage(unk_xyz, kn_xyz, unk_feats, kn_feats, layers):
    d2 = jnp.sum((unk_xyz[:, :, None, :] - kn_xyz[:, None, :, :]) ** 2, axis=-1)
    neg, idx = lax.top_k(-d2, 3)
    w = 1.0 / (-neg + 1e-8)
    w = w / jnp.sum(w, axis=2, keepdims=True)
    gathered = jax.vmap(lambda f, i: f[:, i])(kn_feats, idx)
    interp = jnp.sum(gathered * w[:, None, :, :], axis=-1)
    x = jnp.concatenate([interp, unk_feats], axis=1)
    for lp in layers:
        x = _conv_bn(x, lp, relu=True)
    return x


def kernel(xyz, feats, params):
    f0 = jnp.transpose(feats, (0, 2, 1))
    xs, fs = [xyz], [f0]
    for name, npoint, radii, nsamples in _SA:
        nx, nf = _sa_stage(xs[-1], fs[-1], params[name], npoint, radii, nsamples)
        xs.append(nx)
        fs.append(nf)
    f3 = _fp_stage(xs[3], xs[4], fs[3], fs[4], params['fp4'])
    f2 = _fp_stage(xs[2], xs[3], fs[2], f3, params['fp3'])
    f1 = _fp_stage(xs[1], xs[2], fs[1], f2, params['fp2'])
    f0o = _fp_stage(xs[0], xs[1], fs[0], f1, params['fp1'])
    out = _conv_bn(f0o, params['final'], relu=False)
    return jnp.transpose(out, (0, 2, 1))


# speedup vs baseline: 5.1439x; 3.4501x over previous
import functools

import jax
import jax.numpy as jnp
import numpy as np
from jax import lax
from jax.experimental import pallas as pl
from jax.experimental.pallas import tpu as pltpu

from jax.experimental.pallas import tpu_sc as plsc

_B, _N, _IN_CH, _OUT_CH = 2, 4096, 3, 512
_SA = [
    ('sa1', 1024, [0.02, 0.04], [32, 32]),
    ('sa2', 256, [0.04, 0.08], [32, 32]),
    ('sa3', 64, [0.08, 0.16], [32, 32]),
    ('sa4', 16, [0.16, 0.32], [32, 32]),
]


def _fps_body(npoint, n, S, rows_ref, xt_ref, out_ref):
    x2 = xt_ref[0, 0]
    y2 = xt_ref[0, 1]
    z2 = xt_ref[0, 2]
    iota = (lax.broadcasted_iota(jnp.int32, (S, 128), 0) * 128
            + lax.broadcasted_iota(jnp.int32, (S, 128), 1))
    out_ref[0, pl.ds(0, 1), :] = rows_ref[0, pl.ds(0, 1), :]

    def step(i, carry):
        dists, last = carry
        p = rows_ref[0, pl.ds(last, 1), :]
        px = p[0:1, 0:1]
        py = p[0:1, 1:2]
        pz = p[0:1, 2:3]
        dx = x2 - px
        dy = y2 - py
        dz = z2 - pz
        d = (dx * dx + dy * dy) + dz * dz
        dists = jnp.minimum(dists, d)
        m = jnp.max(dists)
        sel = jnp.where(dists == m, iota, jnp.int32(S * 128))
        nxt = jnp.min(sel)
        out_ref[0, pl.ds(i, 1), :] = rows_ref[0, pl.ds(nxt, 1), :]
        return dists, nxt

    dists0 = jnp.full((S, 128), 1e10, jnp.float32)
    lax.fori_loop(1, npoint, step, (dists0, jnp.int32(0)))


def _fps_level(rows, npoint):
    b, n, _ = rows.shape
    S = max(1, n // 128)
    npad = S * 128
    t = jnp.transpose(rows, (0, 2, 1))
    if npad > n:
        t = jnp.concatenate(
            [t, jnp.broadcast_to(t[:, :, 0:1], (b, 3, npad - n))], axis=2)
    xt = t.reshape(b, 3, S, 128)
    return pl.pallas_call(
        functools.partial(_fps_body, npoint, n, S),
        grid=(b,),
        in_specs=[
            pl.BlockSpec((1, n, 3), lambda i: (i, 0, 0)),
            pl.BlockSpec((1, 3, S, 128), lambda i: (i, 0, 0, 0)),
        ],
        out_specs=pl.BlockSpec((1, npoint, 3), lambda i: (i, 0, 0)),
        out_shape=jax.ShapeDtypeStruct((b, npoint, 3), jnp.float32),
    )(rows, xt)


_NTILES = 32


def _bsel_body(n, npoint, ns, T, r1sq, r2sq, xyzt_ref, c_ref, o1_ref, o2_ref):
    b = pl.program_id(0)
    x = xyzt_ref[0, 0:1, :]
    y = xyzt_ref[0, 1:2, :]
    z = xyzt_ref[0, 2:3, :]
    cx = c_ref[0, :, 0:1]
    cy = c_ref[0, :, 1:2]
    cz = c_ref[0, :, 2:3]
    dx = cx - x
    dy = cy - y
    dz = cz - z
    d2 = (dx * dx + dy * dy) + dz * dz
    iota = lax.broadcasted_iota(jnp.int32, (T, n), 1)
    for rsq, o_ref in ((r1sq, o1_ref), (r2sq, o2_ref)):
        keys = jnp.where(d2 < rsq, iota, jnp.int32(n))
        cols = []
        for _ in range(ns):
            m = jnp.min(keys, axis=1, keepdims=True)
            cols.append(m)
            keys = jnp.where(keys == m, jnp.int32(n), keys)
        idx = jnp.concatenate(cols, axis=1)
        idx = jnp.where(idx == n, idx[:, 0:1], idx)
        o_ref[0] = idx + b * n


def _tc_ball_select(xyz, centers, rsq, ns):
    b, n, _ = xyz.shape
    npoint = centers.shape[1]
    T = min(npoint, 256)
    xyzt = jnp.transpose(xyz, (0, 2, 1))
    body = functools.partial(_bsel_body, n, npoint, ns, T,
                             np.float32(rsq[0]), np.float32(rsq[1]))
    outs = pl.pallas_call(
        body,
        grid=(b, npoint // T),
        in_specs=[
            pl.BlockSpec((1, 3, n), lambda i, j: (i, 0, 0)),
            pl.BlockSpec((1, T, 3), lambda i, j: (i, j, 0)),
        ],
        out_specs=[pl.BlockSpec((1, T, ns), lambda i, j: (i, j, 0)),
                   pl.BlockSpec((1, T, ns), lambda i, j: (i, j, 0))],
        out_shape=[jax.ShapeDtypeStruct((b, npoint, ns), jnp.int32),
                   jax.ShapeDtypeStruct((b, npoint, ns), jnp.int32)],
    )(xyzt, centers)
    return outs


def _sc_gather_body(rows_total, d, ipt, ck, table_hbm, idx1_hbm, idx2_hbm,
                    out1_hbm, out2_hbm, iv1, iv2, rows, sem):
    wid = lax.axis_index("s") * 2 + lax.axis_index("c")
    base = wid * ipt
    pltpu.sync_copy(idx1_hbm.at[pl.ds(base, ipt)], iv1)
    pltpu.sync_copy(idx2_hbm.at[pl.ds(base, ipt)], iv2)

    def chunk(r, carry):
        pltpu.async_copy(table_hbm.at[iv1.at[pl.ds(r * ck, ck)]],
                         rows, sem).wait()
        pltpu.sync_copy(rows, out1_hbm.at[pl.ds(base + r * ck, ck)])
        pltpu.async_copy(table_hbm.at[iv2.at[pl.ds(r * ck, ck)]],
                         rows, sem).wait()
        pltpu.sync_copy(rows, out2_hbm.at[pl.ds(base + r * ck, ck)])
        return carry

    lax.fori_loop(0, ipt // ck, chunk, jnp.int32(0), unroll=False)


def _sc_gather2(table, idx1, idx2):
    rows_total = idx1.shape[0]
    d = table.shape[1]
    ipt = rows_total // _NTILES
    ck = min(128, ipt)
    mesh = plsc.VectorSubcoreMesh(core_axis_name="c", subcore_axis_name="s")
    body = functools.partial(_sc_gather_body, rows_total, d, ipt, ck)
    f = pl.kernel(
        body,
        out_type=[jax.ShapeDtypeStruct((rows_total, d), jnp.float32),
                  jax.ShapeDtypeStruct((rows_total, d), jnp.float32)],
        mesh=mesh,
        scratch_types=[
            pltpu.VMEM((ipt,), jnp.int32),
            pltpu.VMEM((ipt,), jnp.int32),
            pltpu.VMEM((ck, d), jnp.float32),
            pltpu.SemaphoreType.DMA,
        ],
    )
    return f(table, idx1, idx2)


_EPS = 1e-5


def _mm_stats_body(nt, have_corr, x_ref, w_ref, b_ref, c_ref, wc_ref,
                   y_ref, st_ref, acc_ref):
    t = pl.program_id(0)
    y = jnp.dot(x_ref[...], w_ref[...],
                preferred_element_type=jnp.float32) + b_ref[...]
    if have_corr:
        y = y - jnp.dot(c_ref[...], wc_ref[...],
                        preferred_element_type=jnp.float32)
    y_ref[...] = y

    @pl.when(t == 0)
    def _():
        acc_ref[...] = jnp.zeros_like(acc_ref)

    acc_ref[0:1, :] += jnp.sum(y, axis=0, keepdims=True)
    acc_ref[1:2, :] += jnp.sum(y * y, axis=0, keepdims=True)

    @pl.when(t == nt - 1)
    def _():
        st_ref[...] = acc_ref[...]


def _mm_stats(x, wT, bias, corr=None, wcT=None, tr=512):
    R, K = x.shape
    Co = wT.shape[1]
    nt = R // tr
    have_corr = corr is not None
    if not have_corr:
        corr = jnp.zeros((R, 8), jnp.float32)
        wcT = jnp.zeros((8, Co), jnp.float32)
    cw = corr.shape[1]
    y, st = pl.pallas_call(
        functools.partial(_mm_stats_body, nt, have_corr),
        grid=(nt,),
        in_specs=[
            pl.BlockSpec((tr, K), lambda t: (t, 0)),
            pl.BlockSpec((K, Co), lambda t: (0, 0)),
            pl.BlockSpec((1, Co), lambda t: (0, 0)),
            pl.BlockSpec((tr, cw), lambda t: (t, 0)),
            pl.BlockSpec((cw, Co), lambda t: (0, 0)),
        ],
        out_specs=[pl.BlockSpec((tr, Co), lambda t: (t, 0)),
                   pl.BlockSpec((2, Co), lambda t: (0, 0))],
        out_shape=[jax.ShapeDtypeStruct((R, Co), jnp.float32),
                   jax.ShapeDtypeStruct((2, Co), jnp.float32)],
        scratch_shapes=[pltpu.VMEM((2, Co), jnp.float32)],
    )(x, wT, bias.reshape(1, -1), corr, wcT)
    return y, st


def _bn_coefs(st_ref, g_ref, beta_ref, cnt):
    mean = st_ref[0:1, :] * (1.0 / cnt)
    var = st_ref[1:2, :] * (1.0 / cnt) - mean * mean
    a = g_ref[...] * lax.rsqrt(var + _EPS)
    c = beta_ref[...] - mean * a
    return a, c


def _bn_mm_stats_body(nt, cnt, y1_ref, st1_ref, g_ref, beta_ref, w_ref, b_ref,
                      y2_ref, st2_ref, acc_ref):
    t = pl.program_id(0)
    a, c = _bn_coefs(st1_ref, g_ref, beta_ref, cnt)
    h = jnp.maximum(y1_ref[...] * a + c, 0.0)
    y = jnp.dot(h, w_ref[...], preferred_element_type=jnp.float32) + b_ref[...]
    y2_ref[...] = y

    @pl.when(t == 0)
    def _():
        acc_ref[...] = jnp.zeros_like(acc_ref)

    acc_ref[0:1, :] += jnp.sum(y, axis=0, keepdims=True)
    acc_ref[1:2, :] += jnp.sum(y * y, axis=0, keepdims=True)

    @pl.when(t == nt - 1)
    def _():
        st2_ref[...] = acc_ref[...]


def _bn_mm_stats(y1, st1, g, beta, w2T, b2, tr=512):
    R, C1 = y1.shape
    C2 = w2T.shape[1]
    nt = R // tr
    y2, st2 = pl.pallas_call(
        functools.partial(_bn_mm_stats_body, nt, float(R)),
        grid=(nt,),
        in_specs=[
            pl.BlockSpec((tr, C1), lambda t: (t, 0)),
            pl.BlockSpec((2, C1), lambda t: (0, 0)),
            pl.BlockSpec((1, C1), lambda t: (0, 0)),
            pl.BlockSpec((1, C1), lambda t: (0, 0)),
            pl.BlockSpec((C1, C2), lambda t: (0, 0)),
            pl.BlockSpec((1, C2), lambda t: (0, 0)),
        ],
        out_specs=[pl.BlockSpec((tr, C2), lambda t: (t, 0)),
                   pl.BlockSpec((2, C2), lambda t: (0, 0))],
        out_shape=[jax.ShapeDtypeStruct((R, C2), jnp.float32),
                   jax.ShapeDtypeStruct((2, C2), jnp.float32)],
        scratch_shapes=[pltpu.VMEM((2, C2), jnp.float32)],
    )(y1, st1, g.reshape(1, -1), beta.reshape(1, -1), w2T, b2.reshape(1, -1))
    return y2, st2


def _bn_act_body(cnt, relu, ns, y_ref, st_ref, g_ref, beta_ref, o_ref):
    a, c = _bn_coefs(st_ref, g_ref, beta_ref, cnt)
    o = y_ref[...] * a + c
    if relu:
        o = jnp.maximum(o, 0.0)
    if ns:
        tr, C = y_ref.shape
        o = jnp.max(o.reshape(tr // ns, ns, C), axis=1)
    o_ref[...] = o


def _bn_act(y, st, g, beta, relu=True, ns=None, tr=512):
    R, C = y.shape
    nt = R // tr
    ro = tr // ns if ns else tr
    out = pl.pallas_call(
        functools.partial(_bn_act_body, float(R), relu, ns),
        grid=(nt,),
        in_specs=[
            pl.BlockSpec((tr, C), lambda t: (t, 0)),
            pl.BlockSpec((2, C), lambda t: (0, 0)),
            pl.BlockSpec((1, C), lambda t: (0, 0)),
            pl.BlockSpec((1, C), lambda t: (0, 0)),
        ],
        out_specs=pl.BlockSpec((ro, C), lambda t: (t, 0)),
        out_shape=jax.ShapeDtypeStruct((R // ns if ns else R, C), jnp.float32),
    )(y, st, g.reshape(1, -1), beta.reshape(1, -1))
    return out


def _bn(x, g, beta, axes):
    mean = jnp.mean(x, axis=axes, keepdims=True)
    var = jnp.var(x, axis=axes, keepdims=True)
    shape = [1] * x.ndim
    shape[1] = -1
    return (x - mean) / jnp.sqrt(var + 1e-5) * g.reshape(shape) + beta.reshape(shape)


def _conv_bn(x, p, relu=True):
    y = jnp.einsum('oc,bc...->bo...', p['W'], x) + p['b'].reshape((1, -1) + (1,) * (x.ndim - 2))
    y = _bn(y, p['g'], p['beta'], (0,) + tuple(range(2, x.ndim)))
    return jax.nn.relu(y) if relu else y


def _gather(pts, idx):
    return jax.vmap(lambda p, i: p[i])(pts, idx)


def _ball_query(radius, nsample, xyz, new_xyz):
    d2 = jnp.sum((new_xyz[:, :, None, :] - xyz[:, None, :, :]) ** 2, axis=-1)
    n = xyz.shape[1]
    k = jnp.where(d2 < radius * radius, jnp.arange(n, dtype=jnp.int32)[None, None, :], jnp.int32(n))
    idx = -lax.top_k(-k, nsample)[0]
    first = idx[..., :1]
    return jnp.where(idx >= n, first, idx)


def _round_up(v, m):
    return (v + m - 1) // m * m


def _sa_stage(xyz, feats, branches, npoint, radii, nsamples):
    b, n, _ = xyz.shape
    cin = 3 + feats.shape[1]
    d = _round_up(cin, 128)
    new_xyz = _fps_level(xyz, npoint)
    table = jnp.concatenate(
        [xyz, jnp.transpose(feats, (0, 2, 1)),
         jnp.zeros((b, n, d - cin), jnp.float32)], axis=-1).reshape(b * n, d)
    ns = nsamples[0]
    idx1, idx2 = _tc_ball_select(xyz, new_xyz,
                                 (radii[0] ** 2, radii[1] ** 2), ns)
    rows = _sc_gather2(table, idx1.reshape(-1), idx2.reshape(-1))
    outs = []
    for rws, branch in zip(rows, branches):
        g = rws[:, :cin].reshape(b, npoint, ns, cin)
        g = g.at[..., :3].add(-new_xyz[:, :, None, :])
        g = jnp.transpose(g, (0, 3, 1, 2))
        for lp in branch:
            g = _conv_bn(g, lp, relu=True)
        outs.append(jnp.max(g, axis=3))
    return new_xyz, jnp.concatenate(outs, axis=1)


def _fp_stage(unk_xyz, kn_xyz, unk_feats, kn_feats, layers):
    d2 = jnp.sum((unk_xyz[:, :, None, :] - kn_xyz[:, None, :, :]) ** 2, axis=-1)
    neg, idx = lax.top_k(-d2, 3)
    w = 1.0 / (-neg + 1e-8)
    w = w / jnp.sum(w, axis=2, keepdims=True)
    gathered = jax.vmap(lambda f, i: f[:, i])(kn_feats, idx)
    interp = jnp.sum(gathered * w[:, None, :, :], axis=-1)
    x = jnp.concatenate([interp, unk_feats], axis=1)
    for lp in layers:
        x = _conv_bn(x, lp, relu=True)
    return x


def kernel(xyz, feats, params):
    f0 = jnp.transpose(feats, (0, 2, 1))
    xs, fs = [xyz], [f0]
    for name, npoint, radii, nsamples in _SA:
        nx, nf = _sa_stage(xs[-1], fs[-1], params[name], npoint, radii, nsamples)
        xs.append(nx)
        fs.append(nf)
    f3 = _fp_stage(xs[3], xs[4], fs[3], fs[4], params['fp4'])
    f2 = _fp_stage(xs[2], xs[3], fs[2], f3, params['fp3'])
    f1 = _fp_stage(xs[1], xs[2], fs[1], f2, params['fp2'])
    f0o = _fp_stage(xs[0], xs[1], fs[0], f1, params['fp1'])
    out = _conv_bn(f0o, params['final'], relu=False)
    return jnp.transpose(out, (0, 2, 1))
